# baseline (speedup 1.0000x reference)
import functools

import jax
import jax.numpy as jnp
from jax.experimental import pallas as pl

_LATENT_DIM = 256
_CODE_SIZE = 1024
_BETA = 0.25

_TM = 512


def _body(x_ref, cb_ref, q_ref, idx_ref, loss_ref, *, n_total):
    i = pl.program_id(0)
    x = x_ref[...]
    cb = cb_ref[...]
    scores = jax.lax.dot_general(
        x, cb, (((1,), (1,)), ((), ())),
        preferred_element_type=jnp.float32,
        precision=jax.lax.Precision.DEFAULT,
    )
    minval = jnp.min(scores, axis=1, keepdims=True)
    iota = jax.lax.broadcasted_iota(jnp.int32, scores.shape, 1)
    idx = jnp.min(jnp.where(scores == minval, iota, _CODE_SIZE), axis=1)
    idx_ref[...] = idx
    onehot = (iota == idx[:, None]).astype(jnp.float32)
    q = jax.lax.dot_general(
        onehot, cb, (((1,), (0,)), ((), ())),
        preferred_element_type=jnp.float32,
        precision=jax.lax.Precision.HIGHEST,
    )
    q_ref[...] = q
    d = q - x
    part = jnp.sum(d * d, axis=(0, 1), keepdims=True)

    @pl.when(i == 0)
    def _init():
        loss_ref[...] = jnp.zeros_like(loss_ref)

    loss_ref[...] += part

    @pl.when(i == pl.num_programs(0) - 1)
    def _finish():
        loss_ref[...] = loss_ref[...] * ((1.0 + _BETA) / n_total)


def kernel(x, codebook):
    b, t, d = x.shape
    m = b * t
    xf = x.reshape(m, d)
    grid = m // _TM
    q, idx, loss = pl.pallas_call(
        functools.partial(_body, n_total=float(m * d)),
        grid=(grid,),
        in_specs=[
            pl.BlockSpec((_TM, d), lambda i: (i, 0)),
            pl.BlockSpec((_CODE_SIZE, d), lambda i: (0, 0)),
        ],
        out_specs=[
            pl.BlockSpec((_TM, d), lambda i: (i, 0)),
            pl.BlockSpec((_TM,), lambda i: (i,)),
            pl.BlockSpec((1, 1), lambda i: (0, 0)),
        ],
        out_shape=[
            jax.ShapeDtypeStruct((m, d), jnp.float32),
            jax.ShapeDtypeStruct((m,), jnp.int32),
            jax.ShapeDtypeStruct((1, 1), jnp.float32),
        ],
    )(xf, codebook)
    return (q.reshape(b, t, d), loss.reshape(()), idx.reshape(b, t))


# speedup vs baseline: 1.4550x; 1.4550x over previous
import functools

import jax
import jax.numpy as jnp
from jax.experimental import pallas as pl

_LATENT_DIM = 256
_CODE_SIZE = 1024
_BETA = 0.25

_TM = 512


def _body(x_ref, cb_ref, q_ref, idx_ref, loss_ref, *, n_total):
    i = pl.program_id(0)
    x = x_ref[...]
    cb = cb_ref[...]
    scores = jax.lax.dot_general(
        x, cb, (((1,), (1,)), ((), ())),
        preferred_element_type=jnp.float32,
        precision=jax.lax.Precision.DEFAULT,
    )
    minval = jnp.min(scores, axis=1, keepdims=True)
    iota = jax.lax.broadcasted_iota(jnp.int32, scores.shape, 1)
    idx = jnp.min(jnp.where(scores == minval, iota, _CODE_SIZE), axis=1)
    idx_ref[...] = idx
    onehot = (iota == idx[:, None]).astype(jnp.bfloat16)
    cb_hi = cb.astype(jnp.bfloat16)
    cb_lo = (cb - cb_hi.astype(jnp.float32)).astype(jnp.bfloat16)
    dn = (((1,), (0,)), ((), ()))
    q_hi = jax.lax.dot_general(onehot, cb_hi, dn,
                               preferred_element_type=jnp.float32)
    q_lo = jax.lax.dot_general(onehot, cb_lo, dn,
                               preferred_element_type=jnp.float32)
    q = q_hi + q_lo
    q_ref[...] = q
    d = q - x
    part = jnp.sum(d * d, axis=(0, 1), keepdims=True)

    @pl.when(i == 0)
    def _init():
        loss_ref[...] = jnp.zeros_like(loss_ref)

    loss_ref[...] += part

    @pl.when(i == pl.num_programs(0) - 1)
    def _finish():
        loss_ref[...] = loss_ref[...] * ((1.0 + _BETA) / n_total)


def kernel(x, codebook):
    b, t, d = x.shape
    m = b * t
    xf = x.reshape(m, d)
    grid = m // _TM
    q, idx, loss = pl.pallas_call(
        functools.partial(_body, n_total=float(m * d)),
        grid=(grid,),
        in_specs=[
            pl.BlockSpec((_TM, d), lambda i: (i, 0)),
            pl.BlockSpec((_CODE_SIZE, d), lambda i: (0, 0)),
        ],
        out_specs=[
            pl.BlockSpec((_TM, d), lambda i: (i, 0)),
            pl.BlockSpec((_TM,), lambda i: (i,)),
            pl.BlockSpec((1, 1), lambda i: (0, 0)),
        ],
        out_shape=[
            jax.ShapeDtypeStruct((m, d), jnp.float32),
            jax.ShapeDtypeStruct((m,), jnp.int32),
            jax.ShapeDtypeStruct((1, 1), jnp.float32),
        ],
    )(xf, codebook)
    return (q.reshape(b, t, d), loss.reshape(()), idx.reshape(b, t))
